# Initial kernel scaffold; baseline (speedup 1.0000x reference)
import jax
import jax.numpy as jnp
from jax.experimental import pallas as pl


def kernel(inputs, prod_child_ids, prod_segment_ids, sum_child_ids, sum_segment_ids, input_var_ids, input_params, sum_edge_logits, root_logits):
    raise NotImplementedError("write your pallas kernel here")


# speedup vs baseline: 2.6792x; 2.6792x over previous
import functools

import jax
import jax.numpy as jnp
from jax import lax
from jax.experimental import pallas as pl
from jax.experimental.pallas import tpu as pltpu
from jax.experimental.pallas import tpu_sc as plsc

NV = 64
NCAT = 256
NIN = 16384
NEL = 50000
NPE = 100000
NSN = 2048
NSE = 800000
B = 32

NW = 32
CH = 128

NPE_PAD = 102400
NSE_PAD = 802816
NEL_PAD = 50016
NSN_PAD = 2064

_MESH = plsc.VectorSubcoreMesh(core_axis_name="c", subcore_axis_name="s")


def _input_body(params_ref, var_ref, inpf_ref, logp_ref, fidx_ref):
    i = pl.program_id(0)
    p = params_ref[...]
    m = jnp.max(p, axis=1, keepdims=True)
    s = jnp.sum(jnp.exp(p - m), axis=1, keepdims=True)
    logp_ref[...] = p - m - jnp.log(s)
    var = var_ref[0, 0, :]
    oh = (var[:, None] == lax.broadcasted_iota(jnp.int32, (512, NV), 1))
    vals = jnp.dot(oh.astype(jnp.float32), inpf_ref[...],
                   preferred_element_type=jnp.float32)
    n_idx = i * 512 + lax.broadcasted_iota(jnp.int32, (512, B), 0)
    fidx_ref[...] = n_idx * NCAT + vals.astype(jnp.int32)


def _input_layer(params, var3d, inp_f):
    return pl.pallas_call(
        _input_body,
        grid=(NIN // 512,),
        in_specs=[
            pl.BlockSpec((512, NCAT), lambda i: (i, i * 0)),
            pl.BlockSpec((1, 1, 512), lambda i: (i, i * 0, i * 0)),
            pl.BlockSpec((NV, B), lambda i: (i * 0, i * 0)),
        ],
        out_specs=[
            pl.BlockSpec((512, NCAT), lambda i: (i, i * 0)),
            pl.BlockSpec((512, B), lambda i: (i, i * 0)),
        ],
        out_shape=[
            jax.ShapeDtypeStruct((NIN, NCAT), jnp.float32),
            jax.ShapeDtypeStruct((NIN, B), jnp.int32),
        ],
    )(params, var3d, inp_f)


@functools.partial(
    pl.kernel, mesh=_MESH,
    compiler_params=pltpu.CompilerParams(use_tc_tiling_on_sc=False, needs_layout_passes=False),
    out_type=jax.ShapeDtypeStruct((NIN * B, 1), jnp.float32),
    scratch_types=[
        pltpu.VMEM((CH,), jnp.int32),
        pltpu.VMEM((CH, 1), jnp.float32),
        pltpu.SemaphoreType.DMA,
    ],
)
def _sc_word_gather(table, fidx, out, idx_v, rows_v, sem):
    cid = lax.axis_index("c")
    sid = lax.axis_index("s")
    wid = sid * jnp.int32(2) + cid
    base = wid * jnp.int32(NIN * B // NW)

    def body(c, carry):
        off = base + c * jnp.int32(CH)
        pltpu.sync_copy(fidx.at[pl.ds(off, CH)], idx_v)
        pltpu.async_copy(table.at[idx_v], rows_v, sem).wait()
        pltpu.sync_copy(rows_v, out.at[pl.ds(off, CH)])
        return carry

    lax.fori_loop(jnp.int32(0), jnp.int32(NIN * B // NW // CH), body, None)


@functools.partial(
    pl.kernel, mesh=_MESH,
    compiler_params=pltpu.CompilerParams(use_tc_tiling_on_sc=False, needs_layout_passes=False),
    out_type=jax.ShapeDtypeStruct((2, NEL_PAD, B), jnp.float32),
    scratch_types=[
        pltpu.VMEM((CH,), jnp.int32),
        pltpu.VMEM((CH,), jnp.int32),
        pltpu.VMEM((CH, B), jnp.float32),
        pltpu.VMEM_SHARED((NEL_PAD, B), jnp.float32),
        pltpu.SemaphoreType.DMA,
    ],
)
def _sc_prod(node_vals, pc, ps, zeros_el, out, idx_v, seg_v, rows_v,
             elem_sh, sem):
    cid = lax.axis_index("c")
    sid = lax.axis_index("s")
    wid = sid * jnp.int32(2) + cid
    rows_per_tile = NEL_PAD // 16
    r0 = sid * jnp.int32(rows_per_tile)
    pltpu.sync_copy(zeros_el.at[pl.ds(r0, rows_per_tile)],
                    elem_sh.at[pl.ds(r0, rows_per_tile)])
    plsc.subcore_barrier()

    base = wid * jnp.int32(NPE_PAD // NW)

    def body(c, carry):
        off = base + c * jnp.int32(CH)
        pltpu.sync_copy(pc.at[pl.ds(off, CH)], idx_v)
        pltpu.sync_copy(ps.at[pl.ds(off, CH)], seg_v)
        pltpu.async_copy(node_vals.at[idx_v], rows_v, sem).wait()
        pltpu.sync_copy(rows_v, elem_sh.at[seg_v], add=True)
        return carry

    lax.fori_loop(jnp.int32(0), jnp.int32(NPE_PAD // NW // CH), body, None)

    plsc.subcore_barrier()
    pltpu.sync_copy(elem_sh.at[pl.ds(r0, rows_per_tile)],
                    out.at[cid, pl.ds(r0, rows_per_tile)])


def _comb_body(a_ref, b_ref, o_ref):
    o_ref[...] = jnp.exp(a_ref[...] + b_ref[...])


def _combine_exp(a, b):
    blk = 4176
    return pl.pallas_call(
        _comb_body,
        grid=(NEL_PAD // blk,),
        in_specs=[pl.BlockSpec((blk, B), lambda i: (i, i * 0)),
                  pl.BlockSpec((blk, B), lambda i: (i, i * 0))],
        out_specs=pl.BlockSpec((blk, B), lambda i: (i, i * 0)),
        out_shape=jax.ShapeDtypeStruct((NEL_PAD, B), jnp.float32),
    )(a, b)


@functools.partial(
    pl.kernel, mesh=_MESH,
    compiler_params=pltpu.CompilerParams(use_tc_tiling_on_sc=False, needs_layout_passes=False),
    out_type=[
        jax.ShapeDtypeStruct((2, NSN_PAD, B), jnp.float32),
        jax.ShapeDtypeStruct((2, NSN_PAD, 16), jnp.float32),
    ],
    scratch_types=[
        pltpu.VMEM((CH,), jnp.int32),
        pltpu.VMEM((CH,), jnp.int32),
        pltpu.VMEM((CH,), jnp.float32),
        pltpu.VMEM((CH,), jnp.float32),
        pltpu.VMEM((CH, 16), jnp.float32),
        pltpu.VMEM((CH, B), jnp.float32),
        pltpu.VMEM_SHARED((NSN_PAD, B), jnp.float32),
        pltpu.VMEM_SHARED((NSN_PAD, 16), jnp.float32),
        pltpu.SemaphoreType.DMA,
    ],
)
def _sc_sum(elem_prob, sc_, ss_, sl_, zeros_acc, zeros_z, out_acc, out_z,
            idx_v, seg_v, lg_v, u_v, urep, rows_v, acc_sh, z_sh, sem):
    cid = lax.axis_index("c")
    sid = lax.axis_index("s")
    wid = sid * jnp.int32(2) + cid
    rows_per_tile = NSN_PAD // 16
    r0 = sid * jnp.int32(rows_per_tile)
    pltpu.sync_copy(zeros_acc.at[pl.ds(r0, rows_per_tile)],
                    acc_sh.at[pl.ds(r0, rows_per_tile)])
    pltpu.sync_copy(zeros_z.at[pl.ds(r0, rows_per_tile)],
                    z_sh.at[pl.ds(r0, rows_per_tile)])
    zero16 = jnp.zeros((16,), jnp.float32)
    for e in range(CH):
        urep[e, pl.ds(0, 16)] = zero16
    plsc.subcore_barrier()

    base = wid * jnp.int32(NSE_PAD // NW)
    iota16 = lax.iota(jnp.int32, 16)
    col0 = jnp.zeros((16,), jnp.int32)

    def body(c, carry):
        off = base + c * jnp.int32(CH)
        pltpu.sync_copy(sc_.at[pl.ds(off, CH)], idx_v)
        pltpu.sync_copy(ss_.at[pl.ds(off, CH)], seg_v)
        pltpu.sync_copy(sl_.at[pl.ds(off, CH)], lg_v)
        pltpu.async_copy(elem_prob.at[idx_v], rows_v, sem).wait()
        for g in range(CH // 16):
            u16 = jnp.exp(lg_v[pl.ds(g * 16, 16)])
            u_v[pl.ds(g * 16, 16)] = u16
            plsc.store_scatter(urep, [g * 16 + iota16, col0], u16)
        for e in range(CH):
            ue = plsc.load_gather(u_v, [jnp.full((16,), e, jnp.int32)])
            rows_v[e, pl.ds(0, 16)] = rows_v[e, pl.ds(0, 16)] * ue
            rows_v[e, pl.ds(16, 16)] = rows_v[e, pl.ds(16, 16)] * ue
        pltpu.sync_copy(rows_v, acc_sh.at[seg_v], add=True)
        pltpu.sync_copy(urep, z_sh.at[seg_v], add=True)
        return carry

    lax.fori_loop(jnp.int32(0), jnp.int32(NSE_PAD // NW // CH), body, None)

    plsc.subcore_barrier()
    pltpu.sync_copy(acc_sh.at[pl.ds(r0, rows_per_tile)],
                    out_acc.at[cid, pl.ds(r0, rows_per_tile)])
    pltpu.sync_copy(z_sh.at[pl.ds(r0, rows_per_tile)],
                    out_z.at[cid, pl.ds(r0, rows_per_tile)])


def _root_body(rl_ref, aa_ref, ab_ref, za_ref, zb_ref, o_ref):
    rl = rl_ref[...]
    m = jnp.max(rl)
    rw = jnp.exp(rl - m)
    denom = jnp.sum(rw)
    z = za_ref[:, 0:1] + zb_ref[:, 0:1]
    acc = aa_ref[...] + ab_ref[...]
    p = jnp.where(z > 0.0, acc / jnp.where(z > 0.0, z, 1.0), 0.0)
    s = jnp.dot(rw, p, preferred_element_type=jnp.float32)
    o_ref[...] = jnp.log(s) - jnp.log(denom)


def _root(rl2d, acc_a, acc_b, z_a, z_b):
    return pl.pallas_call(
        _root_body,
        grid=(1,),
        in_specs=[
            pl.BlockSpec((1, NSN), lambda i: (i * 0, i * 0)),
            pl.BlockSpec((NSN, B), lambda i: (i * 0, i * 0)),
            pl.BlockSpec((NSN, B), lambda i: (i * 0, i * 0)),
            pl.BlockSpec((NSN, 16), lambda i: (i * 0, i * 0)),
            pl.BlockSpec((NSN, 16), lambda i: (i * 0, i * 0)),
        ],
        out_specs=pl.BlockSpec((1, B), lambda i: (i * 0, i * 0)),
        out_shape=jax.ShapeDtypeStruct((1, B), jnp.float32),
    )(rl2d, acc_a, acc_b, z_a, z_b)


def _pad_i32(x, n, fill):
    x = x.astype(jnp.int32)
    return jnp.concatenate([x, jnp.full((n - x.shape[0],), fill, jnp.int32)])


def kernel(inputs, prod_child_ids, prod_segment_ids, sum_child_ids,
           sum_segment_ids, input_var_ids, input_params, sum_edge_logits,
           root_logits):
    inp_f = inputs.T.astype(jnp.float32)
    var3d = input_var_ids.astype(jnp.int32).reshape(NIN // 512, 1, 512)
    params = input_params.astype(jnp.float32)

    logp, fidx = _input_layer(params, var3d, inp_f)
    node_vals = _sc_word_gather(logp.reshape(NIN * NCAT, 1),
                                fidx.reshape(NIN * B))
    node_vals = node_vals.reshape(NIN, B)

    pc = _pad_i32(prod_child_ids, NPE_PAD, 0)
    ps = _pad_i32(prod_segment_ids, NPE_PAD, NEL)
    elem_parts = _sc_prod(node_vals, pc, ps,
                          jnp.zeros((NEL_PAD, B), jnp.float32))
    elem_prob = _combine_exp(elem_parts[0], elem_parts[1])

    sc_ = _pad_i32(sum_child_ids, NSE_PAD, 0)
    ss_ = _pad_i32(sum_segment_ids, NSE_PAD, NSN)
    sl_ = jnp.concatenate([sum_edge_logits.astype(jnp.float32),
                           jnp.full((NSE_PAD - NSE,), -1e30, jnp.float32)])
    acc_parts, z_parts = _sc_sum(elem_prob, sc_, ss_, sl_,
                                 jnp.zeros((NSN_PAD, B), jnp.float32),
                                 jnp.zeros((NSN_PAD, 16), jnp.float32))

    rl2d = root_logits.astype(jnp.float32).reshape(1, NSN)
    lls = _root(rl2d, acc_parts[0][:NSN], acc_parts[1][:NSN],
                z_parts[0][:NSN], z_parts[1][:NSN])
    return lls.reshape(B, 1)


# speedup vs baseline: 2.7598x; 1.0301x over previous
import functools

import jax
import jax.numpy as jnp
from jax import lax
from jax.experimental import pallas as pl
from jax.experimental.pallas import tpu as pltpu
from jax.experimental.pallas import tpu_sc as plsc

NV = 64
NCAT = 256
NIN = 16384
NEL = 50000
NPE = 100000
NSN = 2048
NSE = 800000
B = 32

NW = 32
CH = 128

NPE_PAD = 102400
NSE_PAD = 802816
NEL_PAD = 50016
NSN_PAD = 2064

_MESH = plsc.VectorSubcoreMesh(core_axis_name="c", subcore_axis_name="s")


def _input_body(params_ref, var_ref, inpf_ref, logp_ref, fidx_ref):
    i = pl.program_id(0)
    p = params_ref[...]
    m = jnp.max(p, axis=1, keepdims=True)
    s = jnp.sum(jnp.exp(p - m), axis=1, keepdims=True)
    logp_ref[...] = p - m - jnp.log(s)
    var = var_ref[0, 0, :]
    oh = (var[:, None] == lax.broadcasted_iota(jnp.int32, (512, NV), 1))
    vals = jnp.dot(oh.astype(jnp.float32), inpf_ref[...],
                   preferred_element_type=jnp.float32)
    n_idx = i * 512 + lax.broadcasted_iota(jnp.int32, (512, B), 0)
    fidx_ref[...] = n_idx * NCAT + vals.astype(jnp.int32)


def _input_layer(params, var3d, inp_f):
    return pl.pallas_call(
        _input_body,
        grid=(NIN // 512,),
        in_specs=[
            pl.BlockSpec((512, NCAT), lambda i: (i, i * 0)),
            pl.BlockSpec((1, 1, 512), lambda i: (i, i * 0, i * 0)),
            pl.BlockSpec((NV, B), lambda i: (i * 0, i * 0)),
        ],
        out_specs=[
            pl.BlockSpec((512, NCAT), lambda i: (i, i * 0)),
            pl.BlockSpec((512, B), lambda i: (i, i * 0)),
        ],
        out_shape=[
            jax.ShapeDtypeStruct((NIN, NCAT), jnp.float32),
            jax.ShapeDtypeStruct((NIN, B), jnp.int32),
        ],
    )(params, var3d, inp_f)


@functools.partial(
    pl.kernel, mesh=_MESH,
    compiler_params=pltpu.CompilerParams(use_tc_tiling_on_sc=False, needs_layout_passes=False),
    out_type=jax.ShapeDtypeStruct((NIN * B, 1), jnp.float32),
    scratch_types=[
        pltpu.VMEM((NIN * B // NW,), jnp.int32),
        pltpu.VMEM((CH, 1), jnp.float32),
        pltpu.SemaphoreType.DMA,
    ],
)
def _sc_word_gather(table, fidx, out, idx_big, rows_v, sem):
    cid = lax.axis_index("c")
    sid = lax.axis_index("s")
    wid = sid * jnp.int32(2) + cid
    base = wid * jnp.int32(NIN * B // NW)
    pltpu.sync_copy(fidx.at[pl.ds(base, NIN * B // NW)], idx_big)

    def body(c, carry):
        off = base + c * jnp.int32(CH)
        idx_v = idx_big.at[pl.ds(c * jnp.int32(CH), CH)]
        pltpu.async_copy(table.at[idx_v], rows_v, sem).wait()
        pltpu.sync_copy(rows_v, out.at[pl.ds(off, CH)])
        return carry

    lax.fori_loop(jnp.int32(0), jnp.int32(NIN * B // NW // CH), body, None)


@functools.partial(
    pl.kernel, mesh=_MESH,
    compiler_params=pltpu.CompilerParams(use_tc_tiling_on_sc=False, needs_layout_passes=False),
    out_type=jax.ShapeDtypeStruct((2, NEL_PAD, B), jnp.float32),
    scratch_types=[
        pltpu.VMEM((NPE_PAD // NW,), jnp.int32),
        pltpu.VMEM((CH,), jnp.int32),
        pltpu.VMEM((CH, B), jnp.float32),
        pltpu.VMEM_SHARED((NEL_PAD, B), jnp.float32),
        pltpu.SemaphoreType.DMA,
    ],
)
def _sc_prod(node_vals, pc, ps, zeros_el, out, idx_big, seg_v, rows_v,
             elem_sh, sem):
    cid = lax.axis_index("c")
    sid = lax.axis_index("s")
    wid = sid * jnp.int32(2) + cid
    rows_per_tile = NEL_PAD // 16
    r0 = sid * jnp.int32(rows_per_tile)
    pltpu.sync_copy(zeros_el.at[pl.ds(r0, rows_per_tile)],
                    elem_sh.at[pl.ds(r0, rows_per_tile)])
    plsc.subcore_barrier()

    base = wid * jnp.int32(NPE_PAD // NW)
    pltpu.sync_copy(pc.at[pl.ds(base, NPE_PAD // NW)], idx_big)

    def body(c, carry):
        off = base + c * jnp.int32(CH)
        pltpu.sync_copy(ps.at[pl.ds(off, CH)], seg_v)
        idx_v = idx_big.at[pl.ds(c * jnp.int32(CH), CH)]
        pltpu.async_copy(node_vals.at[idx_v], rows_v, sem).wait()
        pltpu.sync_copy(rows_v, elem_sh.at[seg_v], add=True)
        return carry

    lax.fori_loop(jnp.int32(0), jnp.int32(NPE_PAD // NW // CH), body, None)

    plsc.subcore_barrier()
    pltpu.sync_copy(elem_sh.at[pl.ds(r0, rows_per_tile)],
                    out.at[cid, pl.ds(r0, rows_per_tile)])


def _comb_body(a_ref, b_ref, o_ref):
    o_ref[...] = jnp.exp(a_ref[...] + b_ref[...])


def _combine_exp(a, b):
    blk = 4176
    return pl.pallas_call(
        _comb_body,
        grid=(NEL_PAD // blk,),
        in_specs=[pl.BlockSpec((blk, B), lambda i: (i, i * 0)),
                  pl.BlockSpec((blk, B), lambda i: (i, i * 0))],
        out_specs=pl.BlockSpec((blk, B), lambda i: (i, i * 0)),
        out_shape=jax.ShapeDtypeStruct((NEL_PAD, B), jnp.float32),
    )(a, b)


@functools.partial(
    pl.kernel, mesh=_MESH,
    compiler_params=pltpu.CompilerParams(use_tc_tiling_on_sc=False, needs_layout_passes=False),
    out_type=[
        jax.ShapeDtypeStruct((2, NSN_PAD, B), jnp.float32),
        jax.ShapeDtypeStruct((2, NSN_PAD, 16), jnp.float32),
    ],
    scratch_types=[
        pltpu.VMEM((NSE_PAD // NW,), jnp.int32),
        pltpu.VMEM((CH,), jnp.int32),
        pltpu.VMEM((NSE_PAD // NW,), jnp.float32),
        pltpu.VMEM((CH,), jnp.float32),
        pltpu.VMEM((CH, 16), jnp.float32),
        pltpu.VMEM((CH, B), jnp.float32),
        pltpu.VMEM_SHARED((NSN_PAD, B), jnp.float32),
        pltpu.VMEM_SHARED((NSN_PAD, 16), jnp.float32),
        pltpu.SemaphoreType.DMA,
    ],
)
def _sc_sum(elem_prob, sc_, ss_, sl_, zeros_acc, zeros_z, out_acc, out_z,
            idx_big, seg_v, lg_big, u_v, urep, rows_v, acc_sh, z_sh, sem):
    cid = lax.axis_index("c")
    sid = lax.axis_index("s")
    wid = sid * jnp.int32(2) + cid
    rows_per_tile = NSN_PAD // 16
    r0 = sid * jnp.int32(rows_per_tile)
    pltpu.sync_copy(zeros_acc.at[pl.ds(r0, rows_per_tile)],
                    acc_sh.at[pl.ds(r0, rows_per_tile)])
    pltpu.sync_copy(zeros_z.at[pl.ds(r0, rows_per_tile)],
                    z_sh.at[pl.ds(r0, rows_per_tile)])
    zero16 = jnp.zeros((16,), jnp.float32)
    for e in range(CH):
        urep[e, pl.ds(0, 16)] = zero16
    plsc.subcore_barrier()

    base = wid * jnp.int32(NSE_PAD // NW)
    pltpu.sync_copy(sc_.at[pl.ds(base, NSE_PAD // NW)], idx_big)
    pltpu.sync_copy(sl_.at[pl.ds(base, NSE_PAD // NW)], lg_big)
    iota16 = lax.iota(jnp.int32, 16)
    col0 = jnp.zeros((16,), jnp.int32)

    def body(c, carry):
        off = base + c * jnp.int32(CH)
        loc = c * jnp.int32(CH)
        pltpu.sync_copy(ss_.at[pl.ds(off, CH)], seg_v)
        idx_v = idx_big.at[pl.ds(loc, CH)]
        pltpu.async_copy(elem_prob.at[idx_v], rows_v, sem).wait()
        for g in range(CH // 16):
            u16 = jnp.exp(lg_big[pl.ds(loc + g * 16, 16)])
            u_v[pl.ds(g * 16, 16)] = u16
            plsc.store_scatter(urep, [g * 16 + iota16, col0], u16)
        for e in range(CH):
            ue = plsc.load_gather(u_v, [jnp.full((16,), e, jnp.int32)])
            rows_v[e, pl.ds(0, 16)] = rows_v[e, pl.ds(0, 16)] * ue
            rows_v[e, pl.ds(16, 16)] = rows_v[e, pl.ds(16, 16)] * ue
        pltpu.sync_copy(rows_v, acc_sh.at[seg_v], add=True)
        pltpu.sync_copy(urep, z_sh.at[seg_v], add=True)
        return carry

    lax.fori_loop(jnp.int32(0), jnp.int32(NSE_PAD // NW // CH), body, None)

    plsc.subcore_barrier()
    pltpu.sync_copy(acc_sh.at[pl.ds(r0, rows_per_tile)],
                    out_acc.at[cid, pl.ds(r0, rows_per_tile)])
    pltpu.sync_copy(z_sh.at[pl.ds(r0, rows_per_tile)],
                    out_z.at[cid, pl.ds(r0, rows_per_tile)])


def _root_body(rl_ref, aa_ref, ab_ref, za_ref, zb_ref, o_ref):
    rl = rl_ref[...]
    m = jnp.max(rl)
    rw = jnp.exp(rl - m)
    denom = jnp.sum(rw)
    z = za_ref[:, 0:1] + zb_ref[:, 0:1]
    acc = aa_ref[...] + ab_ref[...]
    p = jnp.where(z > 0.0, acc / jnp.where(z > 0.0, z, 1.0), 0.0)
    s = jnp.dot(rw, p, preferred_element_type=jnp.float32)
    o_ref[...] = jnp.log(s) - jnp.log(denom)


def _root(rl2d, acc_a, acc_b, z_a, z_b):
    return pl.pallas_call(
        _root_body,
        grid=(1,),
        in_specs=[
            pl.BlockSpec((1, NSN), lambda i: (i * 0, i * 0)),
            pl.BlockSpec((NSN, B), lambda i: (i * 0, i * 0)),
            pl.BlockSpec((NSN, B), lambda i: (i * 0, i * 0)),
            pl.BlockSpec((NSN, 16), lambda i: (i * 0, i * 0)),
            pl.BlockSpec((NSN, 16), lambda i: (i * 0, i * 0)),
        ],
        out_specs=pl.BlockSpec((1, B), lambda i: (i * 0, i * 0)),
        out_shape=jax.ShapeDtypeStruct((1, B), jnp.float32),
    )(rl2d, acc_a, acc_b, z_a, z_b)


def _pad_i32(x, n, fill):
    x = x.astype(jnp.int32)
    return jnp.concatenate([x, jnp.full((n - x.shape[0],), fill, jnp.int32)])


def kernel(inputs, prod_child_ids, prod_segment_ids, sum_child_ids,
           sum_segment_ids, input_var_ids, input_params, sum_edge_logits,
           root_logits):
    inp_f = inputs.T.astype(jnp.float32)
    var3d = input_var_ids.astype(jnp.int32).reshape(NIN // 512, 1, 512)
    params = input_params.astype(jnp.float32)

    logp, fidx = _input_layer(params, var3d, inp_f)
    node_vals = _sc_word_gather(logp.reshape(NIN * NCAT, 1),
                                fidx.reshape(NIN * B))
    node_vals = node_vals.reshape(NIN, B)

    pc = _pad_i32(prod_child_ids, NPE_PAD, 0)
    ps = _pad_i32(prod_segment_ids, NPE_PAD, NEL)
    elem_parts = _sc_prod(node_vals, pc, ps,
                          jnp.zeros((NEL_PAD, B), jnp.float32))
    elem_prob = _combine_exp(elem_parts[0], elem_parts[1])

    sc_ = _pad_i32(sum_child_ids, NSE_PAD, 0)
    ss_ = _pad_i32(sum_segment_ids, NSE_PAD, NSN)
    sl_ = jnp.concatenate([sum_edge_logits.astype(jnp.float32),
                           jnp.full((NSE_PAD - NSE,), -1e30, jnp.float32)])
    acc_parts, z_parts = _sc_sum(elem_prob, sc_, ss_, sl_,
                                 jnp.zeros((NSN_PAD, B), jnp.float32),
                                 jnp.zeros((NSN_PAD, 16), jnp.float32))

    rl2d = root_logits.astype(jnp.float32).reshape(1, NSN)
    lls = _root(rl2d, acc_parts[0][:NSN], acc_parts[1][:NSN],
                z_parts[0][:NSN], z_parts[1][:NSN])
    return lls.reshape(B, 1)
